# baseline (speedup 1.0000x reference)
import functools

import jax
import jax.numpy as jnp
from jax import lax
from jax.experimental import pallas as pl
from jax.experimental.pallas import tpu as pltpu
from jax.experimental.pallas import tpu_sc as plsc

N = 10000
E = 160000
IN_F = 256
HID = 512
OUT_F = 128

NC = 2
NS = 16
CHUNK = 128
E_PAD = 163840
SINK = N
ACC_ROWS = 10240
ROWS_PER_TILE = ACC_ROWS // NS
WB_CHUNKS = ROWS_PER_TILE // CHUNK
_NCH1 = E_PAD // (NS * CHUNK)
_NCH2 = E_PAD // (NC * NS * CHUNK)

_MESH = plsc.VectorSubcoreMesh(
    core_axis_name="c", subcore_axis_name="s", num_cores=NC, num_subcores=NS)

def _fill(ref, nrows, ncols, value):
    v = jnp.full((16,), value, jnp.float32)

    def row(i, _):
        for j in range(ncols // 16):
            ref[i, pl.ds(j * 16, 16)] = v
        return 0

    lax.fori_loop(0, nrows, row, 0)


def _zero_acc(acc, rows, s, ncols):
    _fill(rows, CHUNK, ncols, 0.0)
    for k in range(WB_CHUNKS):
        pltpu.sync_copy(rows, acc.at[pl.ds(s * ROWS_PER_TILE + k * CHUNK, CHUNK)])


def _writeback(acc, rows, out2d, s):
    for k in range(WB_CHUNKS):
        r = s * ROWS_PER_TILE + k * CHUNK
        pltpu.sync_copy(acc.at[pl.ds(r, CHUNK)], rows)
        pltpu.sync_copy(rows, out2d.at[pl.ds(r, CHUNK)])


def _sc_deg_body(dst3, out, idx_d, ones_b, rows, acc, sem0):
    c = lax.axis_index("c")
    s = lax.axis_index("s")
    _zero_acc(acc, rows, s, 128)
    _fill(ones_b, CHUNK, 128, 1.0)
    w = c * NS + s
    pltpu.sync_copy(dst3.at[w], idx_d)
    plsc.subcore_barrier()

    def fire(i, _):
        pltpu.async_copy(ones_b, acc.at[idx_d.at[i]], sem0, add=True)
        return 0

    lax.fori_loop(0, _NCH2, fire, 0)

    def drain(i, _):
        pltpu.make_async_copy(ones_b, acc.at[idx_d.at[0]], sem0).wait()
        return 0

    lax.fori_loop(0, _NCH2, drain, 0)
    plsc.subcore_barrier()

    @pl.when(c == 0)
    def _():
        _writeback(acc, rows, out.at[0], s)

    @pl.when(c == 1)
    def _():
        _writeback(acc, rows, out.at[1], s)


_sc_deg = pl.kernel(
    _sc_deg_body,
    out_type=jax.ShapeDtypeStruct((NC, ACC_ROWS, 128), jnp.float32),
    mesh=_MESH,
    scratch_types=[
        pltpu.VMEM((_NCH2, CHUNK), jnp.int32),
        pltpu.VMEM((CHUNK, 128), jnp.float32),
        pltpu.VMEM((CHUNK, 128), jnp.float32),
        pltpu.VMEM_SHARED((ACC_ROWS, 128), jnp.float32),
        pltpu.SemaphoreType.DMA,
    ],
)


def _pipelined_agg(table, idx_s, idx_d, rows0, rows1, acc, sg0, sg1, ss0,
                   ss1, nch):
    pltpu.async_copy(table.at[idx_s.at[0]], rows0, sg0)
    pltpu.async_copy(table.at[idx_s.at[1]], rows1, sg1)

    def body(j, _):
        k0 = 2 * j
        k1 = k0 + 1
        pltpu.make_async_copy(table.at[idx_s.at[0]], rows0, sg0).wait()
        pltpu.async_copy(rows0, acc.at[idx_d.at[k0]], ss0, add=True)
        pltpu.make_async_copy(table.at[idx_s.at[0]], rows1, sg1).wait()
        pltpu.async_copy(rows1, acc.at[idx_d.at[k1]], ss1, add=True)

        @pl.when(j < nch // 2 - 1)
        def _():
            pltpu.make_async_copy(rows0, acc.at[idx_d.at[0]], ss0).wait()
            pltpu.async_copy(table.at[idx_s.at[k0 + 2]], rows0, sg0)
            pltpu.make_async_copy(rows1, acc.at[idx_d.at[0]], ss1).wait()
            pltpu.async_copy(table.at[idx_s.at[k1 + 2]], rows1, sg1)
        return 0

    lax.fori_loop(0, nch // 2, body, 0)
    pltpu.make_async_copy(rows0, acc.at[idx_d.at[0]], ss0).wait()
    pltpu.make_async_copy(rows1, acc.at[idx_d.at[0]], ss1).wait()


def _sc_agg_feat_body(z1a, z1b, src3, dst3, out, idx_s, idx_d, rows0, rows1,
                      acc, sg0, sg1, ss0, ss1):
    c = lax.axis_index("c")
    s = lax.axis_index("s")
    _zero_acc(acc, rows0, s, 128)
    plsc.subcore_barrier()

    def run(table):
        for p in range(_NCH1 // _NCH2):
            pltpu.sync_copy(src3.at[s, pl.ds(p * _NCH2, _NCH2)], idx_s)
            pltpu.sync_copy(dst3.at[s, pl.ds(p * _NCH2, _NCH2)], idx_d)
            _pipelined_agg(table, idx_s, idx_d, rows0, rows1, acc, sg0,
                           sg1, ss0, ss1, _NCH2)

    @pl.when(c == 0)
    def _():
        run(z1a)

    @pl.when(c == 1)
    def _():
        run(z1b)

    plsc.subcore_barrier()

    @pl.when(c == 0)
    def _():
        _writeback(acc, rows0, out.at[0], s)

    @pl.when(c == 1)
    def _():
        _writeback(acc, rows0, out.at[1], s)


_sc_agg_feat = pl.kernel(
    _sc_agg_feat_body,
    out_type=jax.ShapeDtypeStruct((NC, ACC_ROWS, 128), jnp.float32),
    mesh=_MESH,
    scratch_types=[
        pltpu.VMEM((_NCH2, CHUNK), jnp.int32),
        pltpu.VMEM((_NCH2, CHUNK), jnp.int32),
        pltpu.VMEM((CHUNK, 128), jnp.float32),
        pltpu.VMEM((CHUNK, 128), jnp.float32),
        pltpu.VMEM_SHARED((ACC_ROWS, 128), jnp.float32),
        pltpu.SemaphoreType.DMA,
        pltpu.SemaphoreType.DMA,
        pltpu.SemaphoreType.DMA,
        pltpu.SemaphoreType.DMA,
    ],
)


def _sc_agg_edge_body(z2a, z2b, src3, dst3, out, idx_s, idx_d, rows0, rows1,
                      acc, sg0, sg1, ss0, ss1):
    c = lax.axis_index("c")
    s = lax.axis_index("s")
    _zero_acc(acc, rows0, s, 128)
    w = c * NS + s
    pltpu.sync_copy(src3.at[w], idx_s)
    pltpu.sync_copy(dst3.at[w], idx_d)
    plsc.subcore_barrier()

    @pl.when(c == 0)
    def _():
        _pipelined_agg(z2a, idx_s, idx_d, rows0, rows1, acc, sg0, sg1,
                       ss0, ss1, _NCH2)

    @pl.when(c == 1)
    def _():
        _pipelined_agg(z2b, idx_s, idx_d, rows0, rows1, acc, sg0, sg1,
                       ss0, ss1, _NCH2)

    plsc.subcore_barrier()

    @pl.when(c == 0)
    def _():
        _writeback(acc, rows0, out.at[0], s)

    @pl.when(c == 1)
    def _():
        _writeback(acc, rows0, out.at[1], s)


_sc_agg_edge = pl.kernel(
    _sc_agg_edge_body,
    out_type=jax.ShapeDtypeStruct((NC, ACC_ROWS, 128), jnp.float32),
    mesh=_MESH,
    scratch_types=[
        pltpu.VMEM((_NCH2, CHUNK), jnp.int32),
        pltpu.VMEM((_NCH2, CHUNK), jnp.int32),
        pltpu.VMEM((CHUNK, 128), jnp.float32),
        pltpu.VMEM((CHUNK, 128), jnp.float32),
        pltpu.VMEM_SHARED((ACC_ROWS, 128), jnp.float32),
        pltpu.SemaphoreType.DMA,
        pltpu.SemaphoreType.DMA,
        pltpu.SemaphoreType.DMA,
        pltpu.SemaphoreType.DMA,
    ],
)


_RB = 1000


def _tc_prep_body(x_ref, degp_ref, z1a_ref, z1b_ref, dinv_ref):
    deg = 1.0 + degp_ref[0, :, 0:1] + degp_ref[1, :, 0:1]
    dinv = lax.rsqrt(deg)
    z = x_ref[...] * dinv
    z1a_ref[...] = z[:, :128]
    z1b_ref[...] = z[:, 128:]
    dinv_ref[...] = dinv


def _tc_prep(x, degp):
    return pl.pallas_call(
        _tc_prep_body,
        grid=(N // _RB,),
        in_specs=[
            pl.BlockSpec((_RB, IN_F), lambda b: (b, 0)),
            pl.BlockSpec((NC, _RB, 128), lambda b: (0, b, 0)),
        ],
        out_specs=[
            pl.BlockSpec((_RB, 128), lambda b: (b, 0)),
            pl.BlockSpec((_RB, 128), lambda b: (b, 0)),
            pl.BlockSpec((_RB, 1), lambda b: (b, 0)),
        ],
        out_shape=[
            jax.ShapeDtypeStruct((N, 128), jnp.float32),
            jax.ShapeDtypeStruct((N, 128), jnp.float32),
            jax.ShapeDtypeStruct((N, 1), jnp.float32),
        ],
    )(x, degp)


def _tc_mid_body(s1_ref, z1a_ref, z1b_ref, dinv_ref, W1_ref, b1_ref, W2_ref,
                 z2_ref, z2b_ref):
    dinv = dinv_ref[...]
    a0 = (s1_ref[0] + z1a_ref[...]) * dinv
    a1 = (s1_ref[1] + z1b_ref[...]) * dinv
    agg = jnp.concatenate([a0, a1], axis=1)
    h = jnp.maximum(
        jnp.dot(agg, W1_ref[...], preferred_element_type=jnp.float32)
        + b1_ref[...], 0.0)
    hw = jnp.dot(h, W2_ref[...], preferred_element_type=jnp.float32)
    z2 = hw * dinv
    z2_ref[...] = z2
    z2b_ref[...] = z2


def _tc_mid(s1, z1a, z1b, dinv, W1, b1, W2):
    return pl.pallas_call(
        _tc_mid_body,
        grid=(N // _RB,),
        in_specs=[
            pl.BlockSpec((NC, _RB, 128), lambda b: (0, b, 0)),
            pl.BlockSpec((_RB, 128), lambda b: (b, 0)),
            pl.BlockSpec((_RB, 128), lambda b: (b, 0)),
            pl.BlockSpec((_RB, 1), lambda b: (b, 0)),
            pl.BlockSpec((IN_F, HID), lambda b: (0, 0)),
            pl.BlockSpec((1, HID), lambda b: (0, 0)),
            pl.BlockSpec((HID, OUT_F), lambda b: (0, 0)),
        ],
        out_specs=[
            pl.BlockSpec((_RB, OUT_F), lambda b: (b, 0)),
            pl.BlockSpec((_RB, OUT_F), lambda b: (b, 0)),
        ],
        out_shape=[
            jax.ShapeDtypeStruct((N, OUT_F), jnp.float32),
            jax.ShapeDtypeStruct((N, OUT_F), jnp.float32),
        ],
    )(s1, z1a, z1b, dinv, W1, b1, W2)


def _tc_final_body(s2_ref, z2_ref, dinv_ref, b2_ref, out_ref):
    out_ref[...] = ((s2_ref[0] + s2_ref[1] + z2_ref[...]) * dinv_ref[...]
                    + b2_ref[...])


def _tc_final(s2, z2, dinv, b2):
    return pl.pallas_call(
        _tc_final_body,
        grid=(N // _RB,),
        in_specs=[
            pl.BlockSpec((NC, _RB, OUT_F), lambda b: (0, b, 0)),
            pl.BlockSpec((_RB, OUT_F), lambda b: (b, 0)),
            pl.BlockSpec((_RB, 1), lambda b: (b, 0)),
            pl.BlockSpec((1, OUT_F), lambda b: (0, 0)),
        ],
        out_specs=pl.BlockSpec((_RB, OUT_F), lambda b: (b, 0)),
        out_shape=jax.ShapeDtypeStruct((N, OUT_F), jnp.float32),
    )(s2, z2, dinv, b2)


@jax.jit
def kernel(x, edge_index, W1, b1, W2, b2):
    src = edge_index[0].astype(jnp.int32)
    dst = edge_index[1].astype(jnp.int32)
    pad = E_PAD - E
    pad_idx = jnp.arange(pad, dtype=jnp.int32)
    src_p = jnp.concatenate([src, pad_idx % N])
    dst_p = jnp.concatenate([dst, SINK + pad_idx % (ACC_ROWS - N)])
    src1 = src_p.reshape(NS, _NCH1, CHUNK)
    dst1 = dst_p.reshape(NS, _NCH1, CHUNK)
    src2 = src_p.reshape(NC * NS, _NCH2, CHUNK)
    dst2 = dst_p.reshape(NC * NS, _NCH2, CHUNK)

    degp = _sc_deg(dst2)
    z1a, z1b, dinv = _tc_prep(x, degp)
    s1 = _sc_agg_feat(z1a, z1b, src1, dst1)
    z2, z2b = _tc_mid(s1, z1a, z1b, dinv, W1, b1.reshape(1, HID), W2)
    s2 = _sc_agg_edge(z2, z2b, src2, dst2)
    return _tc_final(s2, z2, dinv, b2.reshape(1, OUT_F))


# speedup vs baseline: 1.1978x; 1.1978x over previous
import functools

import jax
import jax.numpy as jnp
from jax import lax
from jax.experimental import pallas as pl
from jax.experimental.pallas import tpu as pltpu
from jax.experimental.pallas import tpu_sc as plsc

N = 10000
E = 160000
IN_F = 256
HID = 512
OUT_F = 128

NC = 2
NS = 16
CHUNK = 128
E_PAD = 163840
SINK = N
ACC_ROWS = 10240
ROWS_PER_TILE = ACC_ROWS // NS
WB_CHUNKS = ROWS_PER_TILE // CHUNK
_NCH1 = E_PAD // (NS * CHUNK)
_NCH2 = E_PAD // (NC * NS * CHUNK)

_MESH = plsc.VectorSubcoreMesh(
    core_axis_name="c", subcore_axis_name="s", num_cores=NC, num_subcores=NS)

def _fill(ref, nrows, ncols, value):
    v = jnp.full((16,), value, jnp.float32)

    def row(i, _):
        for j in range(ncols // 16):
            ref[i, pl.ds(j * 16, 16)] = v
        return 0

    lax.fori_loop(0, nrows, row, 0)


def _zero_acc(acc, rows, s, ncols):
    _fill(rows, CHUNK, ncols, 0.0)
    for k in range(WB_CHUNKS):
        pltpu.sync_copy(rows, acc.at[pl.ds(s * ROWS_PER_TILE + k * CHUNK, CHUNK)])


def _writeback(acc, rows, out2d, s):
    for k in range(WB_CHUNKS):
        r = s * ROWS_PER_TILE + k * CHUNK
        pltpu.sync_copy(acc.at[pl.ds(r, CHUNK)], rows)
        pltpu.sync_copy(rows, out2d.at[pl.ds(r, CHUNK)])


def _sc_deg_body(dst3, out, idx_d, ones_b, rows, acc, sem0):
    c = lax.axis_index("c")
    s = lax.axis_index("s")
    _zero_acc(acc, rows, s, 128)
    _fill(ones_b, CHUNK, 128, 1.0)
    w = c * NS + s
    pltpu.sync_copy(dst3.at[w], idx_d)
    plsc.subcore_barrier()

    def fire(i, _):
        pltpu.async_copy(ones_b, acc.at[idx_d.at[i]], sem0, add=True)
        return 0

    lax.fori_loop(0, _NCH2, fire, 0)

    def drain(i, _):
        pltpu.make_async_copy(ones_b, acc.at[idx_d.at[0]], sem0).wait()
        return 0

    lax.fori_loop(0, _NCH2, drain, 0)
    plsc.subcore_barrier()

    @pl.when(c == 0)
    def _():
        _writeback(acc, rows, out.at[0], s)

    @pl.when(c == 1)
    def _():
        _writeback(acc, rows, out.at[1], s)


_sc_deg = pl.kernel(
    _sc_deg_body,
    out_type=jax.ShapeDtypeStruct((NC, ACC_ROWS, 128), jnp.float32),
    mesh=_MESH,
    scratch_types=[
        pltpu.VMEM((_NCH2, CHUNK), jnp.int32),
        pltpu.VMEM((CHUNK, 128), jnp.float32),
        pltpu.VMEM((CHUNK, 128), jnp.float32),
        pltpu.VMEM_SHARED((ACC_ROWS, 128), jnp.float32),
        pltpu.SemaphoreType.DMA,
    ],
)


def _pipelined_agg(table, idx_s, idx_d, rows0, rows1, acc, sg0, sg1, ss0,
                   ss1, nch):
    pltpu.async_copy(table.at[idx_s.at[0]], rows0, sg0)

    def body(j, _):
        k0 = 2 * j
        k1 = k0 + 1
        pltpu.async_copy(table.at[idx_s.at[k1]], rows1, sg1)
        pltpu.make_async_copy(table.at[idx_s.at[0]], rows0, sg0).wait()
        pltpu.sync_copy(rows0, acc.at[idx_d.at[k0]], add=True)

        @pl.when(j < nch // 2 - 1)
        def _():
            pltpu.async_copy(table.at[idx_s.at[k0 + 2]], rows0, sg0)

        pltpu.make_async_copy(table.at[idx_s.at[0]], rows1, sg1).wait()
        pltpu.sync_copy(rows1, acc.at[idx_d.at[k1]], add=True)
        return 0

    lax.fori_loop(0, nch // 2, body, 0)


def _sc_agg_feat_body(z1a, z1b, src3, dst3, out, idx_s, idx_d, rows0, rows1,
                      acc, sg0, sg1, ss0, ss1):
    c = lax.axis_index("c")
    s = lax.axis_index("s")
    _zero_acc(acc, rows0, s, 128)
    plsc.subcore_barrier()

    def run(table):
        for p in range(_NCH1 // _NCH2):
            pltpu.sync_copy(src3.at[s, pl.ds(p * _NCH2, _NCH2)], idx_s)
            pltpu.sync_copy(dst3.at[s, pl.ds(p * _NCH2, _NCH2)], idx_d)
            _pipelined_agg(table, idx_s, idx_d, rows0, rows1, acc, sg0,
                           sg1, ss0, ss1, _NCH2)

    @pl.when(c == 0)
    def _():
        run(z1a)

    @pl.when(c == 1)
    def _():
        run(z1b)

    plsc.subcore_barrier()

    @pl.when(c == 0)
    def _():
        _writeback(acc, rows0, out.at[0], s)

    @pl.when(c == 1)
    def _():
        _writeback(acc, rows0, out.at[1], s)


_sc_agg_feat = pl.kernel(
    _sc_agg_feat_body,
    out_type=jax.ShapeDtypeStruct((NC, ACC_ROWS, 128), jnp.float32),
    mesh=_MESH,
    scratch_types=[
        pltpu.VMEM((_NCH2, CHUNK), jnp.int32),
        pltpu.VMEM((_NCH2, CHUNK), jnp.int32),
        pltpu.VMEM((CHUNK, 128), jnp.float32),
        pltpu.VMEM((CHUNK, 128), jnp.float32),
        pltpu.VMEM_SHARED((ACC_ROWS, 128), jnp.float32),
        pltpu.SemaphoreType.DMA,
        pltpu.SemaphoreType.DMA,
        pltpu.SemaphoreType.DMA,
        pltpu.SemaphoreType.DMA,
    ],
)


def _sc_agg_edge_body(z2a, z2b, src3, dst3, out, idx_s, idx_d, rows0, rows1,
                      acc, sg0, sg1, ss0, ss1):
    c = lax.axis_index("c")
    s = lax.axis_index("s")
    _zero_acc(acc, rows0, s, 128)
    w = c * NS + s
    pltpu.sync_copy(src3.at[w], idx_s)
    pltpu.sync_copy(dst3.at[w], idx_d)
    plsc.subcore_barrier()

    @pl.when(c == 0)
    def _():
        _pipelined_agg(z2a, idx_s, idx_d, rows0, rows1, acc, sg0, sg1,
                       ss0, ss1, _NCH2)

    @pl.when(c == 1)
    def _():
        _pipelined_agg(z2b, idx_s, idx_d, rows0, rows1, acc, sg0, sg1,
                       ss0, ss1, _NCH2)

    plsc.subcore_barrier()

    @pl.when(c == 0)
    def _():
        _writeback(acc, rows0, out.at[0], s)

    @pl.when(c == 1)
    def _():
        _writeback(acc, rows0, out.at[1], s)


_sc_agg_edge = pl.kernel(
    _sc_agg_edge_body,
    out_type=jax.ShapeDtypeStruct((NC, ACC_ROWS, 128), jnp.float32),
    mesh=_MESH,
    scratch_types=[
        pltpu.VMEM((_NCH2, CHUNK), jnp.int32),
        pltpu.VMEM((_NCH2, CHUNK), jnp.int32),
        pltpu.VMEM((CHUNK, 128), jnp.float32),
        pltpu.VMEM((CHUNK, 128), jnp.float32),
        pltpu.VMEM_SHARED((ACC_ROWS, 128), jnp.float32),
        pltpu.SemaphoreType.DMA,
        pltpu.SemaphoreType.DMA,
        pltpu.SemaphoreType.DMA,
        pltpu.SemaphoreType.DMA,
    ],
)


_RB = 1000


def _tc_prep_body(x_ref, degp_ref, z1a_ref, z1b_ref, dinv_ref):
    deg = 1.0 + degp_ref[0, :, 0:1] + degp_ref[1, :, 0:1]
    dinv = lax.rsqrt(deg)
    z = x_ref[...] * dinv
    z1a_ref[...] = z[:, :128]
    z1b_ref[...] = z[:, 128:]
    dinv_ref[...] = dinv


def _tc_prep(x, degp):
    return pl.pallas_call(
        _tc_prep_body,
        grid=(N // _RB,),
        in_specs=[
            pl.BlockSpec((_RB, IN_F), lambda b: (b, 0)),
            pl.BlockSpec((NC, _RB, 128), lambda b: (0, b, 0)),
        ],
        out_specs=[
            pl.BlockSpec((_RB, 128), lambda b: (b, 0)),
            pl.BlockSpec((_RB, 128), lambda b: (b, 0)),
            pl.BlockSpec((_RB, 1), lambda b: (b, 0)),
        ],
        out_shape=[
            jax.ShapeDtypeStruct((N, 128), jnp.float32),
            jax.ShapeDtypeStruct((N, 128), jnp.float32),
            jax.ShapeDtypeStruct((N, 1), jnp.float32),
        ],
    )(x, degp)


def _tc_mid_body(s1_ref, z1a_ref, z1b_ref, dinv_ref, W1_ref, b1_ref, W2_ref,
                 z2_ref, z2b_ref):
    dinv = dinv_ref[...]
    a0 = (s1_ref[0] + z1a_ref[...]) * dinv
    a1 = (s1_ref[1] + z1b_ref[...]) * dinv
    agg = jnp.concatenate([a0, a1], axis=1)
    h = jnp.maximum(
        jnp.dot(agg, W1_ref[...], preferred_element_type=jnp.float32)
        + b1_ref[...], 0.0)
    hw = jnp.dot(h, W2_ref[...], preferred_element_type=jnp.float32)
    z2 = hw * dinv
    z2_ref[...] = z2
    z2b_ref[...] = z2


def _tc_mid(s1, z1a, z1b, dinv, W1, b1, W2):
    return pl.pallas_call(
        _tc_mid_body,
        grid=(N // _RB,),
        in_specs=[
            pl.BlockSpec((NC, _RB, 128), lambda b: (0, b, 0)),
            pl.BlockSpec((_RB, 128), lambda b: (b, 0)),
            pl.BlockSpec((_RB, 128), lambda b: (b, 0)),
            pl.BlockSpec((_RB, 1), lambda b: (b, 0)),
            pl.BlockSpec((IN_F, HID), lambda b: (0, 0)),
            pl.BlockSpec((1, HID), lambda b: (0, 0)),
            pl.BlockSpec((HID, OUT_F), lambda b: (0, 0)),
        ],
        out_specs=[
            pl.BlockSpec((_RB, OUT_F), lambda b: (b, 0)),
            pl.BlockSpec((_RB, OUT_F), lambda b: (b, 0)),
        ],
        out_shape=[
            jax.ShapeDtypeStruct((N, OUT_F), jnp.float32),
            jax.ShapeDtypeStruct((N, OUT_F), jnp.float32),
        ],
    )(s1, z1a, z1b, dinv, W1, b1, W2)


def _tc_final_body(s2_ref, z2_ref, dinv_ref, b2_ref, out_ref):
    out_ref[...] = ((s2_ref[0] + s2_ref[1] + z2_ref[...]) * dinv_ref[...]
                    + b2_ref[...])


def _tc_final(s2, z2, dinv, b2):
    return pl.pallas_call(
        _tc_final_body,
        grid=(N // _RB,),
        in_specs=[
            pl.BlockSpec((NC, _RB, OUT_F), lambda b: (0, b, 0)),
            pl.BlockSpec((_RB, OUT_F), lambda b: (b, 0)),
            pl.BlockSpec((_RB, 1), lambda b: (b, 0)),
            pl.BlockSpec((1, OUT_F), lambda b: (0, 0)),
        ],
        out_specs=pl.BlockSpec((_RB, OUT_F), lambda b: (b, 0)),
        out_shape=jax.ShapeDtypeStruct((N, OUT_F), jnp.float32),
    )(s2, z2, dinv, b2)


@jax.jit
def kernel(x, edge_index, W1, b1, W2, b2):
    src = edge_index[0].astype(jnp.int32)
    dst = edge_index[1].astype(jnp.int32)
    pad = E_PAD - E
    pad_idx = jnp.arange(pad, dtype=jnp.int32)
    src_p = jnp.concatenate([src, pad_idx % N])
    dst_p = jnp.concatenate([dst, SINK + pad_idx % (ACC_ROWS - N)])
    src1 = src_p.reshape(NS, _NCH1, CHUNK)
    dst1 = dst_p.reshape(NS, _NCH1, CHUNK)
    src2 = src_p.reshape(NC * NS, _NCH2, CHUNK)
    dst2 = dst_p.reshape(NC * NS, _NCH2, CHUNK)

    degp = _sc_deg(dst2)
    z1a, z1b, dinv = _tc_prep(x, degp)
    s1 = _sc_agg_feat(z1a, z1b, src1, dst1)
    z2, z2b = _tc_mid(s1, z1a, z1b, dinv, W1, b1.reshape(1, HID), W2)
    s2 = _sc_agg_edge(z2, z2b, src2, dst2)
    return _tc_final(s2, z2, dinv, b2.reshape(1, OUT_F))


# speedup vs baseline: 1.1998x; 1.0017x over previous
import jax
import jax.numpy as jnp
from jax import lax
from jax.experimental import pallas as pl
from jax.experimental.pallas import tpu as pltpu
from jax.experimental.pallas import tpu_sc as plsc

N = 10000
E = 160000
IN_F = 256
HID = 512
OUT_F = 128

NC = 2
NS = 16
CHUNK = 128
E_PAD = 163840
SINK = N
ACC_ROWS = 10240
ROWS_PER_TILE = ACC_ROWS // NS
WB_CHUNKS = ROWS_PER_TILE // CHUNK
_NCH1 = E_PAD // (NS * CHUNK)
_NCH2 = E_PAD // (NC * NS * CHUNK)

_MESH = plsc.VectorSubcoreMesh(
    core_axis_name="c", subcore_axis_name="s", num_cores=NC, num_subcores=NS)

def _fill(ref, nrows, ncols, value):
    v = jnp.full((16,), value, jnp.float32)

    def row(i, _):
        for j in range(ncols // 16):
            ref[i, pl.ds(j * 16, 16)] = v
        return 0

    lax.fori_loop(0, nrows, row, 0)


def _zero_acc(acc, rows, s, ncols):
    _fill(rows, CHUNK, ncols, 0.0)
    for k in range(WB_CHUNKS):
        pltpu.sync_copy(rows, acc.at[pl.ds(s * ROWS_PER_TILE + k * CHUNK, CHUNK)])


def _writeback(acc, rows, out2d, s):
    for k in range(WB_CHUNKS):
        r = s * ROWS_PER_TILE + k * CHUNK
        pltpu.sync_copy(acc.at[pl.ds(r, CHUNK)], rows)
        pltpu.sync_copy(rows, out2d.at[pl.ds(r, CHUNK)])


def _sc_deg_body(dst3, out, idx_d, ones_b, rows, acc, sem0):
    c = lax.axis_index("c")
    s = lax.axis_index("s")
    _zero_acc(acc, rows, s, 128)
    _fill(ones_b, CHUNK, 128, 1.0)
    w = c * NS + s
    pltpu.sync_copy(dst3.at[w], idx_d)
    plsc.subcore_barrier()

    def fire(i, _):
        pltpu.async_copy(ones_b, acc.at[idx_d.at[i]], sem0, add=True)
        return 0

    lax.fori_loop(0, _NCH2, fire, 0)

    def drain(i, _):
        pltpu.make_async_copy(ones_b, acc.at[idx_d.at[0]], sem0).wait()
        return 0

    lax.fori_loop(0, _NCH2, drain, 0)
    plsc.subcore_barrier()

    @pl.when(c == 0)
    def _():
        _writeback(acc, rows, out.at[0], s)

    @pl.when(c == 1)
    def _():
        _writeback(acc, rows, out.at[1], s)


_sc_deg = pl.kernel(
    _sc_deg_body,
    out_type=jax.ShapeDtypeStruct((NC, ACC_ROWS, 128), jnp.float32),
    mesh=_MESH,
    scratch_types=[
        pltpu.VMEM((_NCH2, CHUNK), jnp.int32),
        pltpu.VMEM((CHUNK, 128), jnp.float32),
        pltpu.VMEM((CHUNK, 128), jnp.float32),
        pltpu.VMEM_SHARED((ACC_ROWS, 128), jnp.float32),
        pltpu.SemaphoreType.DMA,
    ],
)


def _pipelined_agg(table, idx_s, idx_d, rows0, rows1, acc, sg0, sg1, ss0,
                   ss1, nch):
    pltpu.async_copy(table.at[idx_s.at[0]], rows0, sg0)

    def body(j, _):
        k0 = 2 * j
        k1 = k0 + 1
        pltpu.async_copy(table.at[idx_s.at[k1]], rows1, sg1)
        pltpu.make_async_copy(table.at[idx_s.at[0]], rows0, sg0).wait()
        pltpu.sync_copy(rows0, acc.at[idx_d.at[k0]], add=True)

        @pl.when(j < nch // 2 - 1)
        def _():
            pltpu.async_copy(table.at[idx_s.at[k0 + 2]], rows0, sg0)

        pltpu.make_async_copy(table.at[idx_s.at[0]], rows1, sg1).wait()
        pltpu.sync_copy(rows1, acc.at[idx_d.at[k1]], add=True)
        return 0

    lax.fori_loop(0, nch // 2, body, 0)


def _sc_agg_feat_body(z1a, z1b, src3, dst3, out, idx_s, idx_d, rows0, rows1,
                      acc, sg0, sg1, ss0, ss1):
    c = lax.axis_index("c")
    s = lax.axis_index("s")
    _zero_acc(acc, rows0, s, 128)
    plsc.subcore_barrier()

    def run(table):
        for p in range(_NCH1 // _NCH2):
            pltpu.sync_copy(src3.at[s, pl.ds(p * _NCH2, _NCH2)], idx_s)
            pltpu.sync_copy(dst3.at[s, pl.ds(p * _NCH2, _NCH2)], idx_d)
            _pipelined_agg(table, idx_s, idx_d, rows0, rows1, acc, sg0,
                           sg1, ss0, ss1, _NCH2)

    @pl.when(c == 0)
    def _():
        run(z1a)

    @pl.when(c == 1)
    def _():
        run(z1b)

    plsc.subcore_barrier()

    @pl.when(c == 0)
    def _():
        _writeback(acc, rows0, out.at[0], s)

    @pl.when(c == 1)
    def _():
        _writeback(acc, rows0, out.at[1], s)


_sc_agg_feat = pl.kernel(
    _sc_agg_feat_body,
    out_type=jax.ShapeDtypeStruct((NC, ACC_ROWS, 128), jnp.float32),
    mesh=_MESH,
    scratch_types=[
        pltpu.VMEM((_NCH2, CHUNK), jnp.int32),
        pltpu.VMEM((_NCH2, CHUNK), jnp.int32),
        pltpu.VMEM((CHUNK, 128), jnp.float32),
        pltpu.VMEM((CHUNK, 128), jnp.float32),
        pltpu.VMEM_SHARED((ACC_ROWS, 128), jnp.float32),
        pltpu.SemaphoreType.DMA,
        pltpu.SemaphoreType.DMA,
        pltpu.SemaphoreType.DMA,
        pltpu.SemaphoreType.DMA,
    ],
)


def _sc_agg_edge_body(z2, src3, dst3, out, idx_s, idx_d, rows0, rows1,
                      acc, sg0, sg1, ss0, ss1):
    c = lax.axis_index("c")
    s = lax.axis_index("s")
    _zero_acc(acc, rows0, s, 128)
    w = c * NS + s
    pltpu.sync_copy(src3.at[w], idx_s)
    pltpu.sync_copy(dst3.at[w], idx_d)
    plsc.subcore_barrier()

    _pipelined_agg(z2, idx_s, idx_d, rows0, rows1, acc, sg0, sg1, ss0, ss1,
                   _NCH2)

    plsc.subcore_barrier()

    @pl.when(c == 0)
    def _():
        _writeback(acc, rows0, out.at[0], s)

    @pl.when(c == 1)
    def _():
        _writeback(acc, rows0, out.at[1], s)


_sc_agg_edge = pl.kernel(
    _sc_agg_edge_body,
    out_type=jax.ShapeDtypeStruct((NC, ACC_ROWS, 128), jnp.float32),
    mesh=_MESH,
    scratch_types=[
        pltpu.VMEM((_NCH2, CHUNK), jnp.int32),
        pltpu.VMEM((_NCH2, CHUNK), jnp.int32),
        pltpu.VMEM((CHUNK, 128), jnp.float32),
        pltpu.VMEM((CHUNK, 128), jnp.float32),
        pltpu.VMEM_SHARED((ACC_ROWS, 128), jnp.float32),
        pltpu.SemaphoreType.DMA,
        pltpu.SemaphoreType.DMA,
        pltpu.SemaphoreType.DMA,
        pltpu.SemaphoreType.DMA,
    ],
)


_RB = 1000


def _tc_prep_body(x_ref, degp_ref, z1a_ref, z1b_ref, dinv_ref):
    deg = 1.0 + degp_ref[0, :, 0:1] + degp_ref[1, :, 0:1]
    dinv = lax.rsqrt(deg)
    z = x_ref[...] * dinv
    z1a_ref[...] = z[:, :128]
    z1b_ref[...] = z[:, 128:]
    dinv_ref[...] = dinv


def _tc_prep(x, degp):
    return pl.pallas_call(
        _tc_prep_body,
        grid=(N // _RB,),
        in_specs=[
            pl.BlockSpec((_RB, IN_F), lambda b: (b, 0)),
            pl.BlockSpec((NC, _RB, 128), lambda b: (0, b, 0)),
        ],
        out_specs=[
            pl.BlockSpec((_RB, 128), lambda b: (b, 0)),
            pl.BlockSpec((_RB, 128), lambda b: (b, 0)),
            pl.BlockSpec((_RB, 1), lambda b: (b, 0)),
        ],
        out_shape=[
            jax.ShapeDtypeStruct((N, 128), jnp.float32),
            jax.ShapeDtypeStruct((N, 128), jnp.float32),
            jax.ShapeDtypeStruct((N, 1), jnp.float32),
        ],
    )(x, degp)


def _tc_mid_body(s1_ref, z1a_ref, z1b_ref, dinv_ref, W1_ref, b1_ref, W2_ref,
                 z2_ref):
    dinv = dinv_ref[...]
    a0 = (s1_ref[0] + z1a_ref[...]) * dinv
    a1 = (s1_ref[1] + z1b_ref[...]) * dinv
    agg = jnp.concatenate([a0, a1], axis=1)
    h = jnp.maximum(
        jnp.dot(agg, W1_ref[...], preferred_element_type=jnp.float32)
        + b1_ref[...], 0.0)
    hw = jnp.dot(h, W2_ref[...], preferred_element_type=jnp.float32)
    z2_ref[...] = hw * dinv


def _tc_mid(s1, z1a, z1b, dinv, W1, b1, W2):
    return pl.pallas_call(
        _tc_mid_body,
        grid=(N // _RB,),
        in_specs=[
            pl.BlockSpec((NC, _RB, 128), lambda b: (0, b, 0)),
            pl.BlockSpec((_RB, 128), lambda b: (b, 0)),
            pl.BlockSpec((_RB, 128), lambda b: (b, 0)),
            pl.BlockSpec((_RB, 1), lambda b: (b, 0)),
            pl.BlockSpec((IN_F, HID), lambda b: (0, 0)),
            pl.BlockSpec((1, HID), lambda b: (0, 0)),
            pl.BlockSpec((HID, OUT_F), lambda b: (0, 0)),
        ],
        out_specs=pl.BlockSpec((_RB, OUT_F), lambda b: (b, 0)),
        out_shape=jax.ShapeDtypeStruct((N, OUT_F), jnp.float32),
    )(s1, z1a, z1b, dinv, W1, b1, W2)


def _tc_final_body(s2_ref, z2_ref, dinv_ref, b2_ref, out_ref):
    out_ref[...] = ((s2_ref[0] + s2_ref[1] + z2_ref[...]) * dinv_ref[...]
                    + b2_ref[...])


def _tc_final(s2, z2, dinv, b2):
    return pl.pallas_call(
        _tc_final_body,
        grid=(N // _RB,),
        in_specs=[
            pl.BlockSpec((NC, _RB, OUT_F), lambda b: (0, b, 0)),
            pl.BlockSpec((_RB, OUT_F), lambda b: (b, 0)),
            pl.BlockSpec((_RB, 1), lambda b: (b, 0)),
            pl.BlockSpec((1, OUT_F), lambda b: (0, 0)),
        ],
        out_specs=pl.BlockSpec((_RB, OUT_F), lambda b: (b, 0)),
        out_shape=jax.ShapeDtypeStruct((N, OUT_F), jnp.float32),
    )(s2, z2, dinv, b2)


@jax.jit
def kernel(x, edge_index, W1, b1, W2, b2):
    src = edge_index[0].astype(jnp.int32)
    dst = edge_index[1].astype(jnp.int32)
    pad = E_PAD - E
    pad_idx = jnp.arange(pad, dtype=jnp.int32)
    src_p = jnp.concatenate([src, pad_idx % N])
    dst_p = jnp.concatenate([dst, SINK + pad_idx % (ACC_ROWS - N)])
    src1 = src_p.reshape(NS, _NCH1, CHUNK)
    dst1 = dst_p.reshape(NS, _NCH1, CHUNK)
    src2 = src_p.reshape(NC * NS, _NCH2, CHUNK)
    dst2 = dst_p.reshape(NC * NS, _NCH2, CHUNK)

    degp = _sc_deg(dst2)
    z1a, z1b, dinv = _tc_prep(x, degp)
    s1 = _sc_agg_feat(z1a, z1b, src1, dst1)
    z2 = _tc_mid(s1, z1a, z1b, dinv, W1, b1.reshape(1, HID), W2)
    s2 = _sc_agg_edge(z2, src2, dst2)
    return _tc_final(s2, z2, dinv, b2.reshape(1, OUT_F))
